# baseline (speedup 1.0000x reference)
import functools

import jax
import jax.numpy as jnp
from jax import lax
from jax.experimental import pallas as pl
from jax.experimental.pallas import tpu as pltpu
from jax.experimental.pallas import tpu_sc as plsc

GAMMA = 0.5
DRUG, MIC, DIS = 2048, 512, 1024
CH = 128


def _mm_body(x_ref, w_ref, o_ref):
    o_ref[...] = jnp.dot(x_ref[...], w_ref[...],
                         preferred_element_type=jnp.float32)


def _matmul(x, w, bm):
    n = x.shape[0]
    k = x.shape[1]
    ch = w.shape[1]
    return pl.pallas_call(
        _mm_body,
        grid=(n // bm,),
        in_specs=[
            pl.BlockSpec((bm, k), lambda i: (i, 0)),
            pl.BlockSpec((k, ch), lambda i: (0, 0)),
        ],
        out_specs=pl.BlockSpec((bm, ch), lambda i: (i, 0)),
        out_shape=jax.ShapeDtypeStruct((n, ch), jnp.float32),
    )(x, w)


def _gcn_body(c_ref, h_ref, hblk_ref, degrow_ref, degcol_ref, b_ref, o_ref):
    deg_row = degrow_ref[...]
    dinv_row = jnp.where(deg_row > 0, lax.rsqrt(deg_row), 0.0)
    deg_col = degcol_ref[...]
    dinv_col = jnp.where(deg_col > 0, lax.rsqrt(deg_col), 0.0)
    hd = dinv_row.reshape(-1, 1) * h_ref[...]
    agg = jnp.dot(c_ref[...], hd, preferred_element_type=jnp.float32)
    out = dinv_col * agg + (dinv_col * dinv_col) * hblk_ref[...] + b_ref[...]
    o_ref[...] = jnp.maximum(out, 0.0)


def _gcn_agg(C, h, deg, b, bm):
    n = C.shape[0]
    deg_row = deg.reshape(1, n)
    deg_col = deg.reshape(n, 1)
    b2 = b.reshape(1, CH)
    return pl.pallas_call(
        _gcn_body,
        grid=(n // bm,),
        in_specs=[
            pl.BlockSpec((bm, n), lambda i: (i, 0)),
            pl.BlockSpec((n, CH), lambda i: (0, 0)),
            pl.BlockSpec((bm, CH), lambda i: (i, 0)),
            pl.BlockSpec((1, n), lambda i: (0, 0)),
            pl.BlockSpec((bm, 1), lambda i: (i, 0)),
            pl.BlockSpec((1, CH), lambda i: (0, 0)),
        ],
        out_specs=pl.BlockSpec((bm, CH), lambda i: (i, 0)),
        out_shape=jax.ShapeDtypeStruct((n, CH), jnp.float32),
    )(C, h, h, deg_row, deg_col, b2)


def _gip_body(y_ref, o_ref):
    y = y_ref[...]
    m = y.shape[0]
    mn = jnp.min(y, axis=1, keepdims=True)
    mx = jnp.max(y, axis=1, keepdims=True)
    rng = mx - mn
    rng = jnp.where(rng > 0, rng, 1.0)
    yn = (y - mn) / rng
    s = jnp.sum(yn * yn, axis=1, keepdims=True)
    md = jnp.sum(s) / m
    k = lax.dot_general(yn, yn, (((1,), (1,)), ((), ())),
                        preferred_element_type=jnp.float32) / md
    scol = s / md
    srow = scol.reshape(1, m)
    d = scol + srow - 2.0 * k
    o_ref[...] = jnp.exp(-d * GAMMA)


def _gip(y):
    m = y.shape[0]
    return pl.pallas_call(
        _gip_body,
        out_shape=jax.ShapeDtypeStruct((m, m), jnp.float32),
    )(y)


def _stats_body(k1_ref, k2_ref, diag_ref, minv_ref):
    i = pl.program_id(0)
    bm = k1_ref.shape[0]
    ka = jnp.abs(0.5 * (k1_ref[...] + k2_ref[...]))
    blk_min = jnp.min(jnp.where(ka > 0, ka, jnp.inf))
    prev = jnp.where(i == 0, jnp.inf, minv_ref[0, 0])
    minv_ref[...] = jnp.minimum(prev, blk_min).reshape(1, 1)
    rloc = lax.broadcasted_iota(jnp.int32, (bm, ka.shape[1]), 0)
    cglob = lax.broadcasted_iota(jnp.int32, (bm, ka.shape[1]), 1)
    mask = (rloc + i * bm) == cglob
    diag_ref[...] = jnp.sum(jnp.where(mask, ka, 0.0), axis=1,
                            keepdims=True).reshape(1, bm)


def _kn_stats(k1, k2, bm):
    m = k1.shape[0]
    return pl.pallas_call(
        _stats_body,
        grid=(m // bm,),
        in_specs=[
            pl.BlockSpec((bm, m), lambda i: (i, 0)),
            pl.BlockSpec((bm, m), lambda i: (i, 0)),
        ],
        out_specs=[
            pl.BlockSpec((1, bm), lambda i: (0, i)),
            pl.BlockSpec((1, 1), lambda i: (0, 0)),
        ],
        out_shape=[
            jax.ShapeDtypeStruct((1, m), jnp.float32),
            jax.ShapeDtypeStruct((1, 1), jnp.float32),
        ],
    )(k1, k2)


def _kn_mm_body(k1_ref, k2_ref, diag_ref, minv_ref, a_ref, o_ref):
    ka = jnp.abs(0.5 * (k1_ref[...] + k2_ref[...]))
    kz = jnp.where(ka == 0, minv_ref[0, 0], ka)
    kn = kz / diag_ref[...]
    o_ref[...] = jnp.dot(kn, a_ref[...], preferred_element_type=jnp.float32)


def _kn_matmul(k1, k2, diag, minv, alpha, bm):
    m = k1.shape[0]
    p = alpha.shape[1]
    return pl.pallas_call(
        _kn_mm_body,
        grid=(m // bm,),
        in_specs=[
            pl.BlockSpec((bm, m), lambda i: (i, 0)),
            pl.BlockSpec((bm, m), lambda i: (i, 0)),
            pl.BlockSpec((1, m), lambda i: (0, 0)),
            pl.BlockSpec((1, 1), lambda i: (0, 0)),
            pl.BlockSpec((m, p), lambda i: (0, 0)),
        ],
        out_specs=pl.BlockSpec((bm, p), lambda i: (i, 0)),
        out_shape=jax.ShapeDtypeStruct((m, p), jnp.float32),
    )(k1, k2, diag, minv, alpha)


def _mic_body(k1_ref, k2_ref, a2t_ref, out1_ref, o_ref):
    m = k1_ref.shape[0]
    ka = jnp.abs(0.5 * (k1_ref[...] + k2_ref[...]))
    minv = jnp.min(jnp.where(ka > 0, ka, jnp.inf))
    kz = jnp.where(ka == 0, minv, ka)
    rloc = lax.broadcasted_iota(jnp.int32, (m, m), 0)
    cglob = lax.broadcasted_iota(jnp.int32, (m, m), 1)
    diag = jnp.sum(jnp.where(rloc == cglob, kz, 0.0), axis=1, keepdims=True)
    knt = kz / diag
    out2t = jnp.dot(a2t_ref[...], knt, preferred_element_type=jnp.float32)
    o_ref[...] = (out1_ref[...] + out2t) * 0.5


def _mic_final(mk1, mk2, alpha2t, out1):
    m = mk1.shape[0]
    d = alpha2t.shape[0]
    return pl.pallas_call(
        _mic_body,
        out_shape=jax.ShapeDtypeStruct((d, m), jnp.float32),
    )(mk1, mk2, alpha2t, out1)


_NC, _NS = 2, 16
_ZCH = 16384


def _make_sc_gather_deg(n, e):
    epw = e // (_NC * _NS)
    k = min(2048, epw)
    nch = epw // k
    nz = n // _NS
    mesh = plsc.VectorSubcoreMesh(core_axis_name="c", subcore_axis_name="s")

    @functools.partial(
        pl.kernel, mesh=mesh,
        out_type=[
            jax.ShapeDtypeStruct((_NC, n), jnp.float32),
            jax.ShapeDtypeStruct((e // 128, 128), jnp.float32),
            jax.ShapeDtypeStruct((e // 128, 128), jnp.int32),
        ],
        scratch_types=[
            pltpu.VMEM((16, 128), jnp.int32),
            pltpu.VMEM((16, 128), jnp.int32),
            pltpu.VMEM((16, 128), jnp.int32),
            pltpu.VMEM((16, 128), jnp.int32),
            pltpu.VMEM((16, 128), jnp.float32),
            pltpu.VMEM((nz,), jnp.float32),
            pltpu.VMEM_SHARED((n,), jnp.float32),
            pltpu.SemaphoreType.DMA,
        ],
    )
    def gather_deg(data_hbm, r_hbm, c_hbm, deg_hbm, ew_hbm, didx_hbm,
                   r_v, c_v, idx_v, didx_v, ew_v, zero_v, deg_sh, sem):
        cid = lax.axis_index("c")
        sid = lax.axis_index("s")
        wid = sid * _NC + cid

        def zfill(l, _):
            zero_v[pl.ds(l * 16, 16)] = jnp.zeros((16,), jnp.float32)
            return 0
        lax.fori_loop(0, nz // 16, zfill, 0)
        pltpu.sync_copy(zero_v, deg_sh.at[pl.ds(pl.multiple_of(sid * nz, 8), nz)])
        plsc.subcore_barrier()

        for ch in range(nch):
            b128 = pl.multiple_of((wid * epw + ch * k) // 128, 8)
            pltpu.sync_copy(r_hbm.at[pl.ds(b128, k // 128)], r_v)
            pltpu.sync_copy(c_hbm.at[pl.ds(b128, k // 128)], c_v)
            for j in range(k // 128):
                def body(l, _):
                    rv = r_v[j, pl.ds(l * 16, 16)]
                    cv = c_v[j, pl.ds(l * 16, 16)]
                    idx_v[j, pl.ds(l * 16, 16)] = rv * n + cv
                    didx_v[j, pl.ds(l * 16, 16)] = cv * n + rv
                    return 0
                lax.fori_loop(0, 8, body, 0)
            copies = [pltpu.async_copy(data_hbm.at[idx_v.at[j]], ew_v.at[j],
                                       sem)
                      for j in range(k // 128)]
            for cp in copies:
                cp.wait()
            pltpu.sync_copy(ew_v, ew_hbm.at[pl.ds(b128, k // 128)])
            pltpu.sync_copy(didx_v, didx_hbm.at[pl.ds(b128, k // 128)])
            for j in range(k // 128):
                pltpu.sync_copy(ew_v.at[j], deg_sh.at[c_v.at[j]], add=True)

        plsc.subcore_barrier()
        @pl.when(sid == 0)
        def _():
            pltpu.sync_copy(deg_sh, deg_hbm.at[cid])

    return gather_deg


def _make_sc_build_c(n, e, rs, np_):
    ept = e // _NS
    rows = ept // 128
    words = rs * n // _NS
    rsn = rs * n
    mesh = plsc.VectorSubcoreMesh(core_axis_name="c", subcore_axis_name="s")

    @functools.partial(
        pl.kernel, mesh=mesh,
        out_type=jax.ShapeDtypeStruct((n, n), jnp.float32),
        scratch_types=[
            pltpu.VMEM((16, 128), jnp.int32),
            pltpu.VMEM((16, 128), jnp.float32),
            pltpu.VMEM((16, 128), jnp.int32),
            pltpu.VMEM((16, 128), jnp.float32),
            pltpu.VMEM((_ZCH,), jnp.float32),
            pltpu.VMEM_SHARED((rs * n,), jnp.float32),
            pltpu.SemaphoreType.DMA,
        ],
    )
    def build_c(didx_hbm, ew_hbm, cflat_hbm,
                didx_v, ew_v, lidx_v, val_v, zero_v, c_sh, sem):
        cid = lax.axis_index("c")
        sid = lax.axis_index("s")

        def zfill(l, _):
            zero_v[pl.ds(l * 16, 16)] = jnp.zeros((16,), jnp.float32)
            return 0
        lax.fori_loop(0, _ZCH // 16, zfill, 0)

        for p in range(np_):
            row0 = (p * _NC + cid) * rs
            row0n = row0 * n

            def zcopy(z, _):
                pltpu.sync_copy(
                    zero_v,
                    c_sh.at[pl.ds(pl.multiple_of(sid * words + z * _ZCH, 8),
                                  _ZCH)])
                return 0
            lax.fori_loop(0, words // _ZCH, zcopy, 0)
            plsc.subcore_barrier()

            for ch in range(rows // 16):
                b128 = pl.multiple_of(sid * rows + ch * 16, 8)
                pltpu.sync_copy(didx_hbm.at[pl.ds(b128, 16)], didx_v)
                pltpu.sync_copy(ew_hbm.at[pl.ds(b128, 16)], ew_v)

                def lbody(l, _):
                    dv = didx_v[l // 8, pl.ds((l % 8) * 16, 16)]
                    ev = ew_v[l // 8, pl.ds((l % 8) * 16, 16)]
                    lidx = dv - row0n
                    m = (lidx >= 0) & (lidx < rsn)
                    lidx_v[l // 8, pl.ds((l % 8) * 16, 16)] = jnp.where(
                        m, lidx, dv & 8191)
                    val_v[l // 8, pl.ds((l % 8) * 16, 16)] = jnp.where(
                        m, ev, 0.0)
                    return 0
                lax.fori_loop(0, 128, lbody, 0)
                for j in range(16):
                    pltpu.sync_copy(val_v.at[j], c_sh.at[lidx_v.at[j]],
                                    add=True)
            plsc.subcore_barrier()
            rows_t = rs // _NS
            copies = [
                pltpu.async_copy(
                    c_sh.at[pl.ds(
                        pl.multiple_of((sid * rows_t + i) * n, 8), n)],
                    cflat_hbm.at[row0 + sid * rows_t + i], sem)
                for i in range(rows_t)
            ]
            for cp in copies:
                cp.wait()
            plsc.subcore_barrier()

    return build_c


_SC_PARAMS = {
    2560: dict(rs=640, np_=2),
    3072: dict(rs=512, np_=3),
    1536: dict(rs=768, np_=1),
}


def _build_adjacency(data, edge_index):
    n = data.shape[0]
    e = edge_index.shape[1]
    r2 = edge_index[0].astype(jnp.int32).reshape(e // 128, 128)
    c2 = edge_index[1].astype(jnp.int32).reshape(e // 128, 128)
    deg2, ew2, didx2 = _make_sc_gather_deg(n, e)(data.reshape(-1), r2, c2)
    p = _SC_PARAMS[n]
    C = _make_sc_build_c(n, e, p["rs"], p["np_"])(didx2, ew2)
    deg = deg2[0] + deg2[1] + 1.0
    return C, deg


def kernel(feature, feature_drug_dis, feature_mic_dis, data, data_drug_dis,
           data_mic_dis, edge_index, edge_index_drug_dis, edge_index_mic_dis,
           W1, b1, Wdd, bdd, Wmd, bmd, alpha1, alpha2):
    C1, deg1 = _build_adjacency(data, edge_index)
    C2, deg2 = _build_adjacency(data_drug_dis, edge_index_drug_dis)
    C3, deg3 = _build_adjacency(data_mic_dis, edge_index_mic_dis)
    h1 = _matmul(feature, W1, 512)
    h2 = _matmul(feature_drug_dis, Wdd, 512)
    h3 = _matmul(feature_mic_dis, Wmd, 512)
    H1 = _gcn_agg(C1, h1, deg1, b1, 512)
    Hdd = _gcn_agg(C2, h2, deg2, bdd, 512)
    Hmd = _gcn_agg(C3, h3, deg3, bmd, 512)

    dk1 = _gip(H1[:DRUG])
    mk1 = _gip(H1[DRUG:])
    dk2 = _gip(Hdd[:DRUG])
    mk2 = _gip(Hmd[:MIC])

    diag, minv = _kn_stats(dk1, dk2, 512)
    out1 = _kn_matmul(dk1, dk2, diag, minv, alpha1, 512)
    out = _mic_final(mk1, mk2, alpha2.T, out1)
    return out


# speedup vs baseline: 1.0129x; 1.0129x over previous
import functools

import jax
import jax.numpy as jnp
from jax import lax
from jax.experimental import pallas as pl
from jax.experimental.pallas import tpu as pltpu
from jax.experimental.pallas import tpu_sc as plsc

GAMMA = 0.5
DRUG, MIC, DIS = 2048, 512, 1024
CH = 128


def _mm_body(x_ref, w_ref, o_ref):
    o_ref[...] = jnp.dot(x_ref[...], w_ref[...],
                         preferred_element_type=jnp.float32)


def _matmul(x, w, bm):
    n = x.shape[0]
    k = x.shape[1]
    ch = w.shape[1]
    return pl.pallas_call(
        _mm_body,
        grid=(n // bm,),
        in_specs=[
            pl.BlockSpec((bm, k), lambda i: (i, 0)),
            pl.BlockSpec((k, ch), lambda i: (0, 0)),
        ],
        out_specs=pl.BlockSpec((bm, ch), lambda i: (i, 0)),
        out_shape=jax.ShapeDtypeStruct((n, ch), jnp.float32),
    )(x, w)


def _gcn_body(c_ref, h_ref, hblk_ref, degrow_ref, degcol_ref, b_ref, o_ref):
    deg_row = degrow_ref[...]
    dinv_row = jnp.where(deg_row > 0, lax.rsqrt(deg_row), 0.0)
    deg_col = degcol_ref[...]
    dinv_col = jnp.where(deg_col > 0, lax.rsqrt(deg_col), 0.0)
    hd = dinv_row.reshape(-1, 1) * h_ref[...]
    agg = jnp.dot(c_ref[...], hd, preferred_element_type=jnp.float32)
    out = dinv_col * agg + (dinv_col * dinv_col) * hblk_ref[...] + b_ref[...]
    o_ref[...] = jnp.maximum(out, 0.0)


def _gcn_agg(C, h, deg, b, bm):
    n = C.shape[0]
    deg_row = deg.reshape(1, n)
    deg_col = deg.reshape(n, 1)
    b2 = b.reshape(1, CH)
    return pl.pallas_call(
        _gcn_body,
        grid=(n // bm,),
        in_specs=[
            pl.BlockSpec((bm, n), lambda i: (i, 0)),
            pl.BlockSpec((n, CH), lambda i: (0, 0)),
            pl.BlockSpec((bm, CH), lambda i: (i, 0)),
            pl.BlockSpec((1, n), lambda i: (0, 0)),
            pl.BlockSpec((bm, 1), lambda i: (i, 0)),
            pl.BlockSpec((1, CH), lambda i: (0, 0)),
        ],
        out_specs=pl.BlockSpec((bm, CH), lambda i: (i, 0)),
        out_shape=jax.ShapeDtypeStruct((n, CH), jnp.float32),
    )(C, h, h, deg_row, deg_col, b2)


def _gip_body(y_ref, o_ref):
    y = y_ref[...]
    m = y.shape[0]
    mn = jnp.min(y, axis=1, keepdims=True)
    mx = jnp.max(y, axis=1, keepdims=True)
    rng = mx - mn
    rng = jnp.where(rng > 0, rng, 1.0)
    yn = (y - mn) / rng
    s = jnp.sum(yn * yn, axis=1, keepdims=True)
    md = jnp.sum(s) / m
    k = lax.dot_general(yn, yn, (((1,), (1,)), ((), ())),
                        preferred_element_type=jnp.float32) / md
    scol = s / md
    srow = scol.reshape(1, m)
    d = scol + srow - 2.0 * k
    o_ref[...] = jnp.exp(-d * GAMMA)


def _gip(y):
    m = y.shape[0]
    return pl.pallas_call(
        _gip_body,
        out_shape=jax.ShapeDtypeStruct((m, m), jnp.float32),
    )(y)


def _stats_body(k1_ref, k2_ref, diag_ref, minv_ref):
    i = pl.program_id(0)
    bm = k1_ref.shape[0]
    ka = jnp.abs(0.5 * (k1_ref[...] + k2_ref[...]))
    blk_min = jnp.min(jnp.where(ka > 0, ka, jnp.inf))
    prev = jnp.where(i == 0, jnp.inf, minv_ref[0, 0])
    minv_ref[...] = jnp.minimum(prev, blk_min).reshape(1, 1)
    rloc = lax.broadcasted_iota(jnp.int32, (bm, ka.shape[1]), 0)
    cglob = lax.broadcasted_iota(jnp.int32, (bm, ka.shape[1]), 1)
    mask = (rloc + i * bm) == cglob
    diag_ref[...] = jnp.sum(jnp.where(mask, ka, 0.0), axis=1,
                            keepdims=True).reshape(1, bm)


def _kn_stats(k1, k2, bm):
    m = k1.shape[0]
    return pl.pallas_call(
        _stats_body,
        grid=(m // bm,),
        in_specs=[
            pl.BlockSpec((bm, m), lambda i: (i, 0)),
            pl.BlockSpec((bm, m), lambda i: (i, 0)),
        ],
        out_specs=[
            pl.BlockSpec((1, bm), lambda i: (0, i)),
            pl.BlockSpec((1, 1), lambda i: (0, 0)),
        ],
        out_shape=[
            jax.ShapeDtypeStruct((1, m), jnp.float32),
            jax.ShapeDtypeStruct((1, 1), jnp.float32),
        ],
    )(k1, k2)


def _kn_mm_body(k1_ref, k2_ref, diag_ref, minv_ref, a_ref, o_ref):
    ka = jnp.abs(0.5 * (k1_ref[...] + k2_ref[...]))
    kz = jnp.where(ka == 0, minv_ref[0, 0], ka)
    kn = kz / diag_ref[...]
    o_ref[...] = jnp.dot(kn, a_ref[...], preferred_element_type=jnp.float32)


def _kn_matmul(k1, k2, diag, minv, alpha, bm):
    m = k1.shape[0]
    p = alpha.shape[1]
    return pl.pallas_call(
        _kn_mm_body,
        grid=(m // bm,),
        in_specs=[
            pl.BlockSpec((bm, m), lambda i: (i, 0)),
            pl.BlockSpec((bm, m), lambda i: (i, 0)),
            pl.BlockSpec((1, m), lambda i: (0, 0)),
            pl.BlockSpec((1, 1), lambda i: (0, 0)),
            pl.BlockSpec((m, p), lambda i: (0, 0)),
        ],
        out_specs=pl.BlockSpec((bm, p), lambda i: (i, 0)),
        out_shape=jax.ShapeDtypeStruct((m, p), jnp.float32),
    )(k1, k2, diag, minv, alpha)


def _mic_body(k1_ref, k2_ref, a2t_ref, out1_ref, o_ref):
    m = k1_ref.shape[0]
    ka = jnp.abs(0.5 * (k1_ref[...] + k2_ref[...]))
    minv = jnp.min(jnp.where(ka > 0, ka, jnp.inf))
    kz = jnp.where(ka == 0, minv, ka)
    rloc = lax.broadcasted_iota(jnp.int32, (m, m), 0)
    cglob = lax.broadcasted_iota(jnp.int32, (m, m), 1)
    diag = jnp.sum(jnp.where(rloc == cglob, kz, 0.0), axis=1, keepdims=True)
    knt = kz / diag
    out2t = jnp.dot(a2t_ref[...], knt, preferred_element_type=jnp.float32)
    o_ref[...] = (out1_ref[...] + out2t) * 0.5


def _mic_final(mk1, mk2, alpha2t, out1):
    m = mk1.shape[0]
    d = alpha2t.shape[0]
    return pl.pallas_call(
        _mic_body,
        out_shape=jax.ShapeDtypeStruct((d, m), jnp.float32),
    )(mk1, mk2, alpha2t, out1)


_NC, _NS = 2, 16
_ZCH = 4096


def _make_sc_gather_deg(n, e):
    epw = e // (_NC * _NS)
    k = min(2048, epw)
    nch = epw // k
    nz = n // _NS
    mesh = plsc.VectorSubcoreMesh(core_axis_name="c", subcore_axis_name="s")

    @functools.partial(
        pl.kernel, mesh=mesh,
        out_type=[
            jax.ShapeDtypeStruct((_NC, n), jnp.float32),
            jax.ShapeDtypeStruct((e // 128, 128), jnp.float32),
            jax.ShapeDtypeStruct((e // 128, 128), jnp.int32),
        ],
        scratch_types=[
            pltpu.VMEM((16, 128), jnp.int32),
            pltpu.VMEM((16, 128), jnp.int32),
            pltpu.VMEM((16, 128), jnp.int32),
            pltpu.VMEM((16, 128), jnp.int32),
            pltpu.VMEM((16, 128), jnp.float32),
            pltpu.VMEM((nz,), jnp.float32),
            pltpu.VMEM_SHARED((n,), jnp.float32),
            pltpu.SemaphoreType.DMA,
        ],
    )
    def gather_deg(data_hbm, r_hbm, c_hbm, deg_hbm, ew_hbm, didx_hbm,
                   r_v, c_v, idx_v, didx_v, ew_v, zero_v, deg_sh, sem):
        cid = lax.axis_index("c")
        sid = lax.axis_index("s")
        wid = sid * _NC + cid

        def zfill(l, _):
            zero_v[pl.ds(l * 16, 16)] = jnp.zeros((16,), jnp.float32)
            return 0
        lax.fori_loop(0, nz // 16, zfill, 0)
        pltpu.sync_copy(zero_v, deg_sh.at[pl.ds(pl.multiple_of(sid * nz, 8), nz)])
        plsc.subcore_barrier()

        for ch in range(nch):
            b128 = pl.multiple_of((wid * epw + ch * k) // 128, 8)
            pltpu.sync_copy(r_hbm.at[pl.ds(b128, k // 128)], r_v)
            pltpu.sync_copy(c_hbm.at[pl.ds(b128, k // 128)], c_v)
            for j in range(k // 128):
                def body(l, _):
                    rv = r_v[j, pl.ds(l * 16, 16)]
                    cv = c_v[j, pl.ds(l * 16, 16)]
                    idx_v[j, pl.ds(l * 16, 16)] = rv * n + cv
                    didx_v[j, pl.ds(l * 16, 16)] = cv * n + rv
                    return 0
                lax.fori_loop(0, 8, body, 0)
            copies = [pltpu.async_copy(data_hbm.at[idx_v.at[j]], ew_v.at[j],
                                       sem)
                      for j in range(k // 128)]
            for cp in copies:
                cp.wait()
            pltpu.sync_copy(ew_v, ew_hbm.at[pl.ds(b128, k // 128)])
            pltpu.sync_copy(didx_v, didx_hbm.at[pl.ds(b128, k // 128)])
            for j in range(k // 128):
                pltpu.sync_copy(ew_v.at[j], deg_sh.at[c_v.at[j]], add=True)

        plsc.subcore_barrier()
        @pl.when(sid == 0)
        def _():
            pltpu.sync_copy(deg_sh, deg_hbm.at[cid])

    return gather_deg


def _make_sc_build_c(n, e, rs, np_):
    ept = e // _NS
    rows = ept // 128
    words = rs * n // _NS
    rsn = rs * n
    mesh = plsc.VectorSubcoreMesh(core_axis_name="c", subcore_axis_name="s")

    @functools.partial(
        pl.kernel, mesh=mesh,
        out_type=jax.ShapeDtypeStruct((n, n), jnp.float32),
        scratch_types=[
            pltpu.VMEM((16, 128), jnp.int32),
            pltpu.VMEM((16, 128), jnp.float32),
            pltpu.VMEM((16, 128), jnp.int32),
            pltpu.VMEM((16, 128), jnp.float32),
            pltpu.VMEM((_ZCH,), jnp.float32),
            pltpu.VMEM_SHARED((rs * n,), jnp.float32),
            pltpu.SemaphoreType.DMA,
        ],
    )
    def build_c(didx_hbm, ew_hbm, cflat_hbm,
                didx_v, ew_v, lidx_v, val_v, zero_v, c_sh, sem):
        cid = lax.axis_index("c")
        sid = lax.axis_index("s")

        def zfill(l, _):
            zero_v[pl.ds(l * 16, 16)] = jnp.zeros((16,), jnp.float32)
            return 0
        lax.fori_loop(0, _ZCH // 16, zfill, 0)

        for p in range(np_):
            row0 = (p * _NC + cid) * rs
            row0n = row0 * n

            def zcopy(z, _):
                pltpu.sync_copy(
                    zero_v,
                    c_sh.at[pl.ds(pl.multiple_of(sid * words + z * _ZCH, 8),
                                  _ZCH)])
                return 0
            lax.fori_loop(0, words // _ZCH, zcopy, 0)
            plsc.subcore_barrier()

            for ch in range(rows // 16):
                b128 = pl.multiple_of(sid * rows + ch * 16, 8)
                pltpu.sync_copy(didx_hbm.at[pl.ds(b128, 16)], didx_v)
                pltpu.sync_copy(ew_hbm.at[pl.ds(b128, 16)], ew_v)

                def lbody(l, _):
                    dv = didx_v[l // 8, pl.ds((l % 8) * 16, 16)]
                    ev = ew_v[l // 8, pl.ds((l % 8) * 16, 16)]
                    lidx = dv - row0n
                    m = (lidx >= 0) & (lidx < rsn)
                    lidx_v[l // 8, pl.ds((l % 8) * 16, 16)] = jnp.where(
                        m, lidx, dv & 8191)
                    val_v[l // 8, pl.ds((l % 8) * 16, 16)] = jnp.where(
                        m, ev, 0.0)
                    return 0
                lax.fori_loop(0, 128, lbody, 0)
                for j in range(16):
                    pltpu.sync_copy(val_v.at[j], c_sh.at[lidx_v.at[j]],
                                    add=True)
            plsc.subcore_barrier()
            rows_t = rs // _NS
            copies = [
                pltpu.async_copy(
                    c_sh.at[pl.ds(
                        pl.multiple_of((sid * rows_t + i) * n, 8), n)],
                    cflat_hbm.at[row0 + sid * rows_t + i], sem)
                for i in range(rows_t)
            ]
            for cp in copies:
                cp.wait()
            plsc.subcore_barrier()

    return build_c


_SC_PARAMS = {
    2560: dict(rs=640, np_=2),
    3072: dict(rs=512, np_=3),
    1536: dict(rs=768, np_=1),
}


def _build_adjacency(data, edge_index):
    n = data.shape[0]
    e = edge_index.shape[1]
    r2 = edge_index[0].astype(jnp.int32).reshape(e // 128, 128)
    c2 = edge_index[1].astype(jnp.int32).reshape(e // 128, 128)
    deg2, ew2, didx2 = _make_sc_gather_deg(n, e)(data.reshape(-1), r2, c2)
    p = _SC_PARAMS[n]
    C = _make_sc_build_c(n, e, p["rs"], p["np_"])(didx2, ew2)
    deg = deg2[0] + deg2[1] + 1.0
    return C, deg


def kernel(feature, feature_drug_dis, feature_mic_dis, data, data_drug_dis,
           data_mic_dis, edge_index, edge_index_drug_dis, edge_index_mic_dis,
           W1, b1, Wdd, bdd, Wmd, bmd, alpha1, alpha2):
    C1, deg1 = _build_adjacency(data, edge_index)
    C2, deg2 = _build_adjacency(data_drug_dis, edge_index_drug_dis)
    C3, deg3 = _build_adjacency(data_mic_dis, edge_index_mic_dis)
    h1 = _matmul(feature, W1, 512)
    h2 = _matmul(feature_drug_dis, Wdd, 512)
    h3 = _matmul(feature_mic_dis, Wmd, 512)
    H1 = _gcn_agg(C1, h1, deg1, b1, 512)
    Hdd = _gcn_agg(C2, h2, deg2, bdd, 512)
    Hmd = _gcn_agg(C3, h3, deg3, bmd, 512)

    dk1 = _gip(H1[:DRUG])
    mk1 = _gip(H1[DRUG:])
    dk2 = _gip(Hdd[:DRUG])
    mk2 = _gip(Hmd[:MIC])

    diag, minv = _kn_stats(dk1, dk2, 512)
    out1 = _kn_matmul(dk1, dk2, diag, minv, alpha1, 512)
    out = _mic_final(mk1, mk2, alpha2.T, out1)
    return out
